# Initial kernel scaffold; baseline (speedup 1.0000x reference)
import functools

import jax
import jax.numpy as jnp
from jax import lax
from jax.experimental import pallas as pl
from jax.experimental.pallas import tpu as pltpu
from jax.experimental.pallas import tpu_sc as plsc

N_NODES = 10000
D = 128
N_EDGES = 320000
N_LABEL = 100000

NC = 2
NS = 16
NW = NC * NS
EBLK = 128

EPW = ((N_EDGES // NW) + EBLK - 1) // EBLK * EBLK
NBLK_E = EPW // EBLK
E_PAD = EPW * NW

N_ACC = 10048
ZROWS = N_ACC // NS
OROWS = N_NODES // NS
TRASH = N_NODES

LPW = ((N_LABEL // NW) + EBLK - 1) // EBLK * EBLK
NBLK_L = LPW // EBLK
L_PAD = LPW * NW

_mesh = plsc.VectorSubcoreMesh(core_axis_name="c", subcore_axis_name="s")


def _sc_hist(dstp, ones, zeros16):
    @functools.partial(
        pl.kernel,
        mesh=_mesh,
        out_type=jax.ShapeDtypeStruct((NC, N_NODES, 16), jnp.float32),
        scratch_types=[
            pltpu.VMEM((1, EBLK), jnp.int32),
            pltpu.VMEM((EBLK, 16), jnp.float32),
            pltpu.VMEM_SHARED((N_ACC, 16), jnp.float32),
        ],
    )
    def k(dst_hbm, ones_hbm, z_hbm, out_hbm, di, ones_v, hist):
        c = lax.axis_index("c")
        s = lax.axis_index("s")
        wid = c * NS + s
        pltpu.sync_copy(ones_hbm, ones_v)
        pltpu.sync_copy(z_hbm.at[pl.ds(s * ZROWS, ZROWS)],
                        hist.at[pl.ds(s * ZROWS, ZROWS)])
        plsc.subcore_barrier()
        base = wid * EPW

        @pl.loop(0, NBLK_E)
        def _(b):
            off = base + b * EBLK
            pltpu.sync_copy(dst_hbm.at[pl.ds(off, EBLK)], di.at[0])
            pltpu.sync_copy(ones_v, hist.at[di.at[0]], add=True)

        plsc.subcore_barrier()
        pltpu.sync_copy(hist.at[pl.ds(s * OROWS, OROWS)],
                        out_hbm.at[c].at[pl.ds(s * OROWS, OROWS)])

    return k(dstp, ones, zeros16)


def _sc_segsum(table, srcp, dstp, zeros):
    @functools.partial(
        pl.kernel,
        mesh=_mesh,
        out_type=jax.ShapeDtypeStruct((NC, N_NODES, D), jnp.float32),
        scratch_types=[
            pltpu.VMEM((1, EBLK), jnp.int32),
            pltpu.VMEM((1, EBLK), jnp.int32),
            pltpu.VMEM((EBLK, D), jnp.float32),
            pltpu.VMEM_SHARED((N_ACC, D), jnp.float32),
            pltpu.SemaphoreType.DMA,
        ],
    )
    def k(tab_hbm, src_hbm, dst_hbm, z_hbm, out_hbm, si, di, rows, acc, sem):
        c = lax.axis_index("c")
        s = lax.axis_index("s")
        wid = c * NS + s
        pltpu.sync_copy(z_hbm.at[pl.ds(s * ZROWS, ZROWS)],
                        acc.at[pl.ds(s * ZROWS, ZROWS)])
        plsc.subcore_barrier()
        base = wid * EPW

        @pl.loop(0, NBLK_E)
        def _(b):
            off = base + b * EBLK
            pltpu.sync_copy(src_hbm.at[pl.ds(off, EBLK)], si.at[0])
            pltpu.sync_copy(dst_hbm.at[pl.ds(off, EBLK)], di.at[0])
            pltpu.async_copy(tab_hbm.at[si.at[0]], rows, sem).wait()
            pltpu.sync_copy(rows, acc.at[di.at[0]], add=True)

        plsc.subcore_barrier()
        pltpu.sync_copy(acc.at[pl.ds(s * OROWS, OROWS)],
                        out_hbm.at[c].at[pl.ds(s * OROWS, OROWS)])

    return k(table, srcp, dstp, zeros)


def _sc_decode_gather(z, lsrc, ldst):
    out_t = jax.ShapeDtypeStruct((L_PAD, D), jnp.float32)

    @functools.partial(
        pl.kernel,
        mesh=_mesh,
        out_type=(out_t, out_t),
        scratch_types=[
            pltpu.VMEM((1, EBLK), jnp.int32),
            pltpu.VMEM((EBLK, D), jnp.float32),
            pltpu.SemaphoreType.DMA,
        ],
    )
    def k(z_hbm, s_hbm, d_hbm, os_hbm, od_hbm, ii, rows, sem):
        c = lax.axis_index("c")
        s = lax.axis_index("s")
        wid = c * NS + s
        base = wid * LPW

        @pl.loop(0, NBLK_L)
        def _(b):
            off = base + b * EBLK
            pltpu.sync_copy(s_hbm.at[pl.ds(off, EBLK)], ii.at[0])
            pltpu.async_copy(z_hbm.at[ii.at[0]], rows, sem).wait()
            pltpu.sync_copy(rows, os_hbm.at[pl.ds(off, EBLK)])
            pltpu.sync_copy(d_hbm.at[pl.ds(off, EBLK)], ii.at[0])
            pltpu.async_copy(z_hbm.at[ii.at[0]], rows, sem).wait()
            pltpu.sync_copy(rows, od_hbm.at[pl.ds(off, EBLK)])

    return k(z, lsrc, ldst)


def _tc_mm1(x, W1):
    def body(x_ref, w_ref, o_ref):
        o_ref[...] = jnp.dot(x_ref[...], w_ref[...],
                             preferred_element_type=jnp.float32)

    return pl.pallas_call(
        body, out_shape=jax.ShapeDtypeStruct((N_NODES, D), jnp.float32),
    )(x, W1)


def _tc_prep(hist, xw1):
    def body(h_ref, xw_ref, dinv_ref, y_ref):
        deg = h_ref[0, :, 0] + h_ref[1, :, 0] + 1.0
        dinv = lax.rsqrt(deg)
        dinv_ref[...] = dinv[:, None]
        y_ref[...] = xw_ref[...] * dinv[:, None]

    return pl.pallas_call(
        body,
        out_shape=(jax.ShapeDtypeStruct((N_NODES, 1), jnp.float32),
                   jax.ShapeDtypeStruct((N_NODES, D), jnp.float32)),
    )(hist, xw1)


def _tc_mid(acc1, y1, dinv, b1, W2):
    def body(a_ref, y_ref, d_ref, b_ref, w_ref, o_ref):
        dinv = d_ref[...]
        h = jnp.maximum(dinv * (a_ref[0] + a_ref[1] + y_ref[...]) + b_ref[...],
                        0.0)
        o_ref[...] = jnp.dot(h, w_ref[...],
                             preferred_element_type=jnp.float32) * dinv

    return pl.pallas_call(
        body, out_shape=jax.ShapeDtypeStruct((N_NODES, D), jnp.float32),
    )(acc1, y1, dinv, b1, W2)


def _tc_fin(acc2, y2, dinv, b2):
    def body(a_ref, y_ref, d_ref, b_ref, o_ref):
        o_ref[...] = d_ref[...] * (a_ref[0] + a_ref[1] + y_ref[...]) + b_ref[...]

    return pl.pallas_call(
        body, out_shape=jax.ShapeDtypeStruct((N_NODES, D), jnp.float32),
    )(acc2, y2, dinv, b2)


def _tc_dot(zs, zd):
    RB = 12800

    def body(a_ref, b_ref, o_ref):
        o_ref[...] = jnp.sum(a_ref[...] * b_ref[...], axis=1)

    return pl.pallas_call(
        body,
        grid=(L_PAD // RB,),
        in_specs=[pl.BlockSpec((RB, D), lambda i: (i, 0)),
                  pl.BlockSpec((RB, D), lambda i: (i, 0))],
        out_specs=pl.BlockSpec((RB,), lambda i: (i,)),
        out_shape=jax.ShapeDtypeStruct((L_PAD,), jnp.float32),
    )(zs, zd)


def kernel(x, edge_index, edge_label_index, W1, b1, W2, b2):
    src = edge_index[0].astype(jnp.int32)
    dst = edge_index[1].astype(jnp.int32)
    lsrc = edge_label_index[0].astype(jnp.int32)
    ldst = edge_label_index[1].astype(jnp.int32)

    epad = E_PAD - N_EDGES
    srcp = jnp.concatenate([src, jnp.zeros((epad,), jnp.int32)])
    dstp = jnp.concatenate([dst, jnp.full((epad,), TRASH, jnp.int32)])
    lpad = L_PAD - N_LABEL
    lsrcp = jnp.concatenate([lsrc, jnp.zeros((lpad,), jnp.int32)])
    ldstp = jnp.concatenate([ldst, jnp.zeros((lpad,), jnp.int32)])

    ones16 = jnp.ones((EBLK, 16), jnp.float32)
    zeros16 = jnp.zeros((N_ACC, 16), jnp.float32)
    zeros128 = jnp.zeros((N_ACC, D), jnp.float32)

    hist = _sc_hist(dstp, ones16, zeros16)
    xw1 = _tc_mm1(x, W1)
    dinv, y1 = _tc_prep(hist, xw1)
    acc1 = _sc_segsum(y1, srcp, dstp, zeros128)
    y2 = _tc_mid(acc1, y1, dinv, b1, W2)
    acc2 = _sc_segsum(y2, srcp, dstp, zeros128)
    z = _tc_fin(acc2, y2, dinv, b2)
    zs, zd = _sc_decode_gather(z, lsrcp, ldstp)
    return _tc_dot(zs, zd)[:N_LABEL]


# speedup vs baseline: 7.5560x; 7.5560x over previous
import functools

import jax
import jax.numpy as jnp
from jax import lax
from jax.experimental import pallas as pl
from jax.experimental.pallas import tpu as pltpu
from jax.experimental.pallas import tpu_sc as plsc

N_NODES = 10000
D = 128
N_EDGES = 320000
N_LABEL = 100000

NC = 2
NS = 16
NW = NC * NS
EBLK = 128

EPW = ((N_EDGES // NW) + EBLK - 1) // EBLK * EBLK
NBLK_E = EPW // EBLK
E_PAD = EPW * NW

N_PAD = 10112
ZROWS = N_PAD // NS
TRASH = N_NODES

LPW = ((N_LABEL // NW) + EBLK - 1) // EBLK * EBLK
NBLK_L = LPW // EBLK
L_PAD = LPW * NW

_mesh = plsc.VectorSubcoreMesh(core_axis_name="c", subcore_axis_name="s")


def _sc_hist(dstp, ones, zeros):
    @functools.partial(
        pl.kernel,
        mesh=_mesh,
        out_type=jax.ShapeDtypeStruct((NC, N_PAD, D), jnp.float32),
        scratch_types=[
            pltpu.VMEM((1, EBLK), jnp.int32),
            pltpu.VMEM((EBLK, D), jnp.float32),
            pltpu.VMEM_SHARED((N_PAD, D), jnp.float32),
        ],
    )
    def k(dst_hbm, ones_hbm, z_hbm, out_hbm, di, ones_v, hist):
        c = lax.axis_index("c")
        s = lax.axis_index("s")
        wid = c * NS + s
        pltpu.sync_copy(ones_hbm, ones_v)
        pltpu.sync_copy(z_hbm.at[pl.ds(s * ZROWS, ZROWS)],
                        hist.at[pl.ds(s * ZROWS, ZROWS)])
        plsc.subcore_barrier()
        base = wid * EPW

        @pl.loop(0, NBLK_E)
        def _(b):
            off = base + b * EBLK
            pltpu.sync_copy(dst_hbm.at[pl.ds(off, EBLK)], di.at[0])
            pltpu.sync_copy(ones_v, hist.at[di.at[0]], add=True)

        plsc.subcore_barrier()
        pltpu.sync_copy(hist.at[pl.ds(s * ZROWS, ZROWS)],
                        out_hbm.at[c].at[pl.ds(s * ZROWS, ZROWS)])

    return k(dstp, ones, zeros)


def _sc_segsum(table, srcp, dstp, zeros):
    @functools.partial(
        pl.kernel,
        mesh=_mesh,
        out_type=jax.ShapeDtypeStruct((NC, N_PAD, D), jnp.float32),
        scratch_types=[
            pltpu.VMEM((1, EBLK), jnp.int32),
            pltpu.VMEM((1, EBLK), jnp.int32),
            pltpu.VMEM((EBLK, D), jnp.float32),
            pltpu.VMEM_SHARED((N_PAD, D), jnp.float32),
            pltpu.SemaphoreType.DMA,
        ],
    )
    def k(tab_hbm, src_hbm, dst_hbm, z_hbm, out_hbm, si, di, rows, acc, sem):
        c = lax.axis_index("c")
        s = lax.axis_index("s")
        wid = c * NS + s
        pltpu.sync_copy(z_hbm.at[pl.ds(s * ZROWS, ZROWS)],
                        acc.at[pl.ds(s * ZROWS, ZROWS)])
        plsc.subcore_barrier()
        base = wid * EPW

        @pl.loop(0, NBLK_E)
        def _(b):
            off = base + b * EBLK
            pltpu.sync_copy(src_hbm.at[pl.ds(off, EBLK)], si.at[0])
            pltpu.sync_copy(dst_hbm.at[pl.ds(off, EBLK)], di.at[0])
            pltpu.async_copy(tab_hbm.at[si.at[0]], rows, sem).wait()
            pltpu.sync_copy(rows, acc.at[di.at[0]], add=True)

        plsc.subcore_barrier()
        pltpu.sync_copy(acc.at[pl.ds(s * ZROWS, ZROWS)],
                        out_hbm.at[c].at[pl.ds(s * ZROWS, ZROWS)])

    return k(table, srcp, dstp, zeros)


def _sc_decode_gather(z, lsrc, ldst):
    out_t = jax.ShapeDtypeStruct((L_PAD, D), jnp.float32)

    @functools.partial(
        pl.kernel,
        mesh=_mesh,
        out_type=(out_t, out_t),
        scratch_types=[
            pltpu.VMEM((1, EBLK), jnp.int32),
            pltpu.VMEM((EBLK, D), jnp.float32),
            pltpu.SemaphoreType.DMA,
        ],
    )
    def k(z_hbm, s_hbm, d_hbm, os_hbm, od_hbm, ii, rows, sem):
        c = lax.axis_index("c")
        s = lax.axis_index("s")
        wid = c * NS + s
        base = wid * LPW

        @pl.loop(0, NBLK_L)
        def _(b):
            off = base + b * EBLK
            pltpu.sync_copy(s_hbm.at[pl.ds(off, EBLK)], ii.at[0])
            pltpu.async_copy(z_hbm.at[ii.at[0]], rows, sem).wait()
            pltpu.sync_copy(rows, os_hbm.at[pl.ds(off, EBLK)])
            pltpu.sync_copy(d_hbm.at[pl.ds(off, EBLK)], ii.at[0])
            pltpu.async_copy(z_hbm.at[ii.at[0]], rows, sem).wait()
            pltpu.sync_copy(rows, od_hbm.at[pl.ds(off, EBLK)])

    return k(z, lsrc, ldst)


def _tc_mm1(x, W1):
    def body(x_ref, w_ref, o_ref):
        o_ref[...] = jnp.dot(x_ref[...], w_ref[...],
                             preferred_element_type=jnp.float32)

    return pl.pallas_call(
        body, out_shape=jax.ShapeDtypeStruct((N_PAD, D), jnp.float32),
    )(x, W1)


def _tc_prep(hist, xw1):
    def body(h_ref, xw_ref, dinv_ref, y_ref):
        deg = h_ref[0, :, 0] + h_ref[1, :, 0] + 1.0
        dinv = lax.rsqrt(deg)
        dinv_ref[...] = dinv[:, None]
        y_ref[...] = xw_ref[...] * dinv[:, None]

    return pl.pallas_call(
        body,
        out_shape=(jax.ShapeDtypeStruct((N_PAD, 1), jnp.float32),
                   jax.ShapeDtypeStruct((N_PAD, D), jnp.float32)),
    )(hist, xw1)


def _tc_mid(acc1, y1, dinv, b1, W2):
    def body(a_ref, y_ref, d_ref, b_ref, w_ref, o_ref):
        dinv = d_ref[...]
        h = jnp.maximum(dinv * (a_ref[0] + a_ref[1] + y_ref[...]) + b_ref[...],
                        0.0)
        o_ref[...] = jnp.dot(h, w_ref[...],
                             preferred_element_type=jnp.float32) * dinv

    return pl.pallas_call(
        body, out_shape=jax.ShapeDtypeStruct((N_PAD, D), jnp.float32),
    )(acc1, y1, dinv, b1, W2)


def _tc_fin(acc2, y2, dinv, b2):
    def body(a_ref, y_ref, d_ref, b_ref, o_ref):
        o_ref[...] = d_ref[...] * (a_ref[0] + a_ref[1] + y_ref[...]) + b_ref[...]

    return pl.pallas_call(
        body, out_shape=jax.ShapeDtypeStruct((N_PAD, D), jnp.float32),
    )(acc2, y2, dinv, b2)


def _tc_dot(zs, zd):
    RB = 12800
    NB = L_PAD // RB

    def body(a_ref, b_ref, o_ref):
        o_ref[...] = jnp.sum(a_ref[...] * b_ref[...], axis=1).reshape(8, RB // 8)

    return pl.pallas_call(
        body,
        grid=(NB,),
        in_specs=[pl.BlockSpec((RB, D), lambda i: (i, 0)),
                  pl.BlockSpec((RB, D), lambda i: (i, 0))],
        out_specs=pl.BlockSpec((8, RB // 8), lambda i: (i, 0)),
        out_shape=jax.ShapeDtypeStruct((NB * 8, RB // 8), jnp.float32),
    )(zs, zd)


def kernel(x, edge_index, edge_label_index, W1, b1, W2, b2):
    src = edge_index[0].astype(jnp.int32)
    dst = edge_index[1].astype(jnp.int32)
    lsrc = edge_label_index[0].astype(jnp.int32)
    ldst = edge_label_index[1].astype(jnp.int32)

    epad = E_PAD - N_EDGES
    srcp = jnp.concatenate([src, jnp.zeros((epad,), jnp.int32)])
    dstp = jnp.concatenate([dst, jnp.full((epad,), TRASH, jnp.int32)])
    lpad = L_PAD - N_LABEL
    lsrcp = jnp.concatenate([lsrc, jnp.zeros((lpad,), jnp.int32)])
    ldstp = jnp.concatenate([ldst, jnp.zeros((lpad,), jnp.int32)])

    xp = jnp.concatenate([x, jnp.zeros((N_PAD - N_NODES, D), jnp.float32)])

    ones = jnp.ones((EBLK, D), jnp.float32)
    zeros128 = jnp.zeros((N_PAD, D), jnp.float32)

    hist = _sc_hist(dstp, ones, zeros128)
    xw1 = _tc_mm1(xp, W1)
    dinv, y1 = _tc_prep(hist, xw1)
    acc1 = _sc_segsum(y1, srcp, dstp, zeros128)
    y2 = _tc_mid(acc1, y1, dinv, b1, W2)
    acc2 = _sc_segsum(y2, srcp, dstp, zeros128)
    z = _tc_fin(acc2, y2, dinv, b2)
    zs, zd = _sc_decode_gather(z, lsrcp, ldstp)
    return _tc_dot(zs, zd).reshape(L_PAD)[:N_LABEL]
